# baseline (speedup 1.0000x reference)
import functools

import jax
import jax.numpy as jnp
from jax import lax
from jax.experimental import pallas as pl
from jax.experimental.pallas import tpu as pltpu
from jax.experimental.pallas import tpu_sc as plsc

N = 10000
D = 128
E = 320000

NC = 2
NS = 16
NW = NC * NS

CHUNK = 128
E_PAD = 327680
NCHUNKS = E_PAD // CHUNK
CPT = NCHUNKS // NW
N_PAD = 10240
RPT = N_PAD // NS
DEG_W = 16
NBUF = 2
LAG = 1
NGRP = CPT // NBUF

_R = 2048
_G = N_PAD // _R


def _sc_mesh():
    return plsc.VectorSubcoreMesh(core_axis_name="c", subcore_axis_name="s")


_SC_PARAMS = pltpu.CompilerParams(use_tc_tiling_on_sc=False)


def _sc_deg(dst2d):
    @functools.partial(
        pl.kernel,
        mesh=_sc_mesh(),
        out_type=jax.ShapeDtypeStruct((NC, N_PAD, DEG_W), jnp.float32),
        scratch_types=[
            pltpu.VMEM((CPT, CHUNK), jnp.int32),
            pltpu.VMEM((CHUNK, DEG_W), jnp.float32),
            pltpu.VMEM((CHUNK, DEG_W), jnp.float32),
            pltpu.VMEM_SHARED((N_PAD, DEG_W), jnp.float32),
            pltpu.SemaphoreType.DMA,
        ],
        compiler_params=_SC_PARAMS,
    )
    def k(dst_hbm, out_hbm, idxb, onesb, zerob, acc, sem):
        c = lax.axis_index("c")
        s = lax.axis_index("s")
        w = c * NS + s

        def fill(i, carry):
            onesb[i, :] = jnp.full((DEG_W,), 1.0, jnp.float32)
            zerob[i, :] = jnp.zeros((DEG_W,), jnp.float32)
            return carry

        lax.fori_loop(0, CHUNK, fill, 0)
        for kk in range(RPT // CHUNK):
            pltpu.sync_copy(zerob, acc.at[pl.ds(s * RPT + kk * CHUNK, CHUNK)])
        pltpu.sync_copy(dst_hbm.at[pl.ds(w * CPT, CPT)], idxb)
        plsc.subcore_barrier()

        def fire(g, carry):
            pltpu.async_copy(onesb, acc.at[idxb.at[g]], sem, add=True)
            return carry

        lax.fori_loop(0, CPT, fire, 0)

        def drain(g, carry):
            pltpu.make_async_copy(onesb, acc.at[idxb.at[g]], sem).wait()
            return carry

        lax.fori_loop(0, CPT, drain, 0)
        plsc.subcore_barrier()
        pltpu.sync_copy(acc.at[pl.ds(s * RPT, RPT)],
                        out_hbm.at[c, pl.ds(s * RPT, RPT)])

    return k(dst2d)


def _sc_agg(hs, src2d, dst2d):
    @functools.partial(
        pl.kernel,
        mesh=_sc_mesh(),
        out_type=jax.ShapeDtypeStruct((NC, N_PAD, D), jnp.float32),
        scratch_types=[
            pltpu.VMEM((2, NBUF, CHUNK), jnp.int32),
            pltpu.VMEM((CPT, CHUNK), jnp.int32),
            pltpu.VMEM((NBUF, CHUNK, D), jnp.float32),
            pltpu.VMEM_SHARED((N_PAD, D), jnp.float32),
            [pltpu.SemaphoreType.DMA] * NBUF,
            pltpu.SemaphoreType.DMA,
        ],
        compiler_params=_SC_PARAMS,
    )
    def k(hs_hbm, src_hbm, dst_hbm, out_hbm, srcs, dstb, rows, acc, sems,
          semi):
        c = lax.axis_index("c")
        s = lax.axis_index("s")
        w = c * NS + s
        base = w * CPT

        def zrow(i, carry):
            for j in range(D // 16):
                rows[0, i, pl.ds(j * 16, 16)] = jnp.zeros((16,), jnp.float32)
            return carry

        lax.fori_loop(0, CHUNK, zrow, 0)
        for kk in range(RPT // CHUNK):
            pltpu.sync_copy(rows.at[0],
                            acc.at[pl.ds(s * RPT + kk * CHUNK, CHUNK)])
        pltpu.sync_copy(dst_hbm.at[pl.ds(base, CPT)], dstb)
        pltpu.sync_copy(src_hbm.at[pl.ds(base, NBUF)], srcs.at[0])
        plsc.subcore_barrier()

        def fire_g(ch, b, idx_row):
            pltpu.async_copy(hs_hbm.at[idx_row], rows.at[b], sems[b])

        def wait_g(b):
            pltpu.make_async_copy(hs_hbm.at[srcs.at[0, 0]], rows.at[b],
                                  sems[b]).wait()

        def fire_s(ch, b):
            pltpu.async_copy(rows.at[b], acc.at[dstb.at[ch]], sems[b],
                             add=True)

        def wait_s(ch, b):
            pltpu.make_async_copy(rows.at[b], acc.at[dstb.at[ch]],
                                  sems[b]).wait()

        pltpu.async_copy(src_hbm.at[pl.ds(base + NBUF, NBUF)], srcs.at[1],
                         semi)
        fire_g(0, 0, srcs.at[0, 0])
        fire_g(1, 1, srcs.at[0, 1])
        wait_g(0)
        fire_s(0, 0)

        def main(r, carry):
            gp = lax.rem(r + 1, 2)
            pltpu.make_async_copy(
                src_hbm.at[pl.ds(base, NBUF)], srcs.at[0], semi).wait()
            ch0 = 2 * r + 1
            wait_s(ch0 - 1, 0)
            fire_g(ch0 + 1, 0, srcs.at[gp, 0])
            wait_g(1)
            fire_s(ch0, 1)
            nxt = jnp.minimum(r + 2, NGRP - 1)
            pltpu.async_copy(
                src_hbm.at[pl.ds(base + nxt * NBUF, NBUF)],
                srcs.at[lax.rem(r, 2)], semi)
            ch1 = 2 * r + 2
            wait_s(ch1 - 1, 1)
            fire_g(ch1 + 1, 1, srcs.at[gp, 1])
            wait_g(0)
            fire_s(ch1, 0)
            return carry

        lax.fori_loop(0, NGRP - 1, main, 0)
        pltpu.make_async_copy(src_hbm.at[pl.ds(base, NBUF)], srcs.at[0],
                              semi).wait()
        wait_s(CPT - 2, 0)
        wait_g(1)
        fire_s(CPT - 1, 1)
        wait_s(CPT - 1, 1)
        plsc.subcore_barrier()
        pltpu.sync_copy(acc.at[pl.ds(s * RPT, RPT)],
                        out_hbm.at[c, pl.ds(s * RPT, RPT)])

    return k(hs, src2d, dst2d)


def _tc1(degp, x_p, W1):
    def body(degp_ref, x_ref, w_ref, hs_ref, dis_ref):
        d16 = degp_ref[0] + degp_ref[1]
        deg = jnp.sum(d16, axis=1, keepdims=True) * (1.0 / DEG_W) + 1.0
        dis = lax.rsqrt(deg)
        h = jnp.dot(x_ref[...], w_ref[...],
                    preferred_element_type=jnp.float32) * dis
        hs_ref[...] = h
        dis_ref[...] = dis

    return pl.pallas_call(
        body,
        grid=(_G,),
        in_specs=[
            pl.BlockSpec((NC, _R, DEG_W), lambda r: (0, r, 0)),
            pl.BlockSpec((_R, D), lambda r: (r, 0)),
            pl.BlockSpec((D, D), lambda r: (0, 0)),
        ],
        out_specs=[
            pl.BlockSpec((_R, D), lambda r: (r, 0)),
            pl.BlockSpec((_R, 1), lambda r: (r, 0)),
        ],
        out_shape=[
            jax.ShapeDtypeStruct((N_PAD, D), jnp.float32),
            jax.ShapeDtypeStruct((N_PAD, 1), jnp.float32),
        ],
    )(degp, x_p, W1)


def _tc_mid(p, hs, dis, b, W2, first):
    def body(p_ref, hs_ref, dis_ref, b_ref, w_ref, out_ref):
        r = pl.program_id(0)
        dis_v = dis_ref[...]
        u = dis_v * (p_ref[0] + p_ref[1] + hs_ref[...]) + b_ref[...]
        if first:
            row = lax.broadcasted_iota(jnp.int32, (_R, 1), 0) + r * _R
            t = jnp.where(row < N, jnp.maximum(u, 0.0), 0.0)
            out_ref[...] = jnp.dot(
                t, w_ref[...], preferred_element_type=jnp.float32) * dis_v
        else:
            out_ref[...] = u

    return pl.pallas_call(
        body,
        grid=(_G,),
        in_specs=[
            pl.BlockSpec((NC, _R, D), lambda r: (0, r, 0)),
            pl.BlockSpec((_R, D), lambda r: (r, 0)),
            pl.BlockSpec((_R, 1), lambda r: (r, 0)),
            pl.BlockSpec((1, D), lambda r: (0, 0)),
            pl.BlockSpec((D, D), lambda r: (0, 0)),
        ],
        out_specs=pl.BlockSpec((_R, D), lambda r: (r, 0)),
        out_shape=jax.ShapeDtypeStruct((N_PAD, D), jnp.float32),
    )(p, hs, dis, b, W2)


def kernel(x, edge_index, W1, b1, W2, b2):
    src = edge_index[0]
    dst = edge_index[1]
    pad = jnp.full((E_PAD - E,), N, jnp.int32)
    src2d = jnp.concatenate([src, pad]).reshape(NCHUNKS, CHUNK)
    dst2d = jnp.concatenate([dst, pad]).reshape(NCHUNKS, CHUNK)
    x_p = jnp.pad(x, ((0, N_PAD - N), (0, 0)))

    degp = _sc_deg(dst2d)
    hs1, dis = _tc1(degp, x_p, W1)
    p = _sc_agg(hs1, src2d, dst2d)
    hs2 = _tc_mid(p, hs1, dis, b1.reshape(1, D), W2, first=True)
    q = _sc_agg(hs2, src2d, dst2d)
    out = _tc_mid(q, hs2, dis, b2.reshape(1, D), W2, first=False)
    return out[:N]


# speedup vs baseline: 1.0034x; 1.0034x over previous
import functools

import jax
import jax.numpy as jnp
from jax import lax
from jax.experimental import pallas as pl
from jax.experimental.pallas import tpu as pltpu
from jax.experimental.pallas import tpu_sc as plsc

N = 10000
D = 128
E = 320000

NC = 2
NS = 16
NW = NC * NS

CHUNK = 128
E_PAD = 327680
NCHUNKS = E_PAD // CHUNK
CPT = NCHUNKS // NW
N_PAD = 10240
RPT = N_PAD // NS
DEG_W = 16
NBUF = 2
LAG = 1
NGRP = CPT // NBUF

_R = 2048
_G = N_PAD // _R


def _sc_mesh():
    return plsc.VectorSubcoreMesh(core_axis_name="c", subcore_axis_name="s")


_SC_PARAMS = pltpu.CompilerParams(use_tc_tiling_on_sc=False)


def _sc_deg(dst2d):
    @functools.partial(
        pl.kernel,
        mesh=_sc_mesh(),
        out_type=jax.ShapeDtypeStruct((NC, N_PAD, DEG_W), jnp.float32),
        scratch_types=[
            pltpu.VMEM((CPT, CHUNK), jnp.int32),
            pltpu.VMEM((CHUNK, DEG_W), jnp.float32),
            pltpu.VMEM((CHUNK, DEG_W), jnp.float32),
            pltpu.VMEM_SHARED((N_PAD, DEG_W), jnp.float32),
            pltpu.SemaphoreType.DMA,
        ],
        compiler_params=_SC_PARAMS,
    )
    def k(dst_hbm, out_hbm, idxb, onesb, zerob, acc, sem):
        c = lax.axis_index("c")
        s = lax.axis_index("s")
        w = c * NS + s

        def fill(i, carry):
            onesb[i, :] = jnp.full((DEG_W,), 1.0, jnp.float32)
            zerob[i, :] = jnp.zeros((DEG_W,), jnp.float32)
            return carry

        lax.fori_loop(0, CHUNK, fill, 0)
        for kk in range(RPT // CHUNK):
            pltpu.sync_copy(zerob, acc.at[pl.ds(s * RPT + kk * CHUNK, CHUNK)])
        pltpu.sync_copy(dst_hbm.at[pl.ds(w * CPT, CPT)], idxb)
        plsc.subcore_barrier()

        def fire(g, carry):
            pltpu.async_copy(onesb, acc.at[idxb.at[g]], sem, add=True)
            return carry

        lax.fori_loop(0, CPT, fire, 0)

        def drain(g, carry):
            pltpu.make_async_copy(onesb, acc.at[idxb.at[g]], sem).wait()
            return carry

        lax.fori_loop(0, CPT, drain, 0)
        plsc.subcore_barrier()
        pltpu.sync_copy(acc.at[pl.ds(s * RPT, RPT)],
                        out_hbm.at[c, pl.ds(s * RPT, RPT)])

    return k(dst2d)


ACH = 64
ACPT = E_PAD // ACH // NW
ANBUF = 4
ALAG = 2
ANGRP = ACPT // ANBUF


def _sc_agg(hs, src2d, dst2d):
    @functools.partial(
        pl.kernel,
        mesh=_sc_mesh(),
        out_type=jax.ShapeDtypeStruct((NC, N_PAD, D), jnp.float32),
        scratch_types=[
            pltpu.VMEM((2, ANBUF, ACH), jnp.int32),
            pltpu.VMEM((ACPT, ACH), jnp.int32),
            pltpu.VMEM((ANBUF, ACH, D), jnp.float32),
            pltpu.VMEM_SHARED((N_PAD, D), jnp.float32),
            [pltpu.SemaphoreType.DMA] * ANBUF,
            pltpu.SemaphoreType.DMA,
        ],
        compiler_params=_SC_PARAMS,
    )
    def k(hs_hbm, src_hbm, dst_hbm, out_hbm, srcs, dstb, rows, acc, sems,
          semi):
        c = lax.axis_index("c")
        s = lax.axis_index("s")
        w = c * NS + s
        base = w * ACPT

        def zrow(i, carry):
            for j in range(D // 16):
                rows[0, i, pl.ds(j * 16, 16)] = jnp.zeros((16,), jnp.float32)
            return carry

        lax.fori_loop(0, ACH, zrow, 0)
        for kk in range(RPT // ACH):
            pltpu.sync_copy(rows.at[0],
                            acc.at[pl.ds(s * RPT + kk * ACH, ACH)])
        pltpu.sync_copy(dst_hbm.at[pl.ds(base, ACPT)], dstb)
        pltpu.sync_copy(src_hbm.at[pl.ds(base, ANBUF)], srcs.at[0])
        plsc.subcore_barrier()

        def fire_g(ch, b, idx_row):
            pltpu.async_copy(hs_hbm.at[idx_row], rows.at[b], sems[b])

        def wait_g(b):
            pltpu.make_async_copy(hs_hbm.at[srcs.at[0, 0]], rows.at[b],
                                  sems[b]).wait()

        def fire_s(ch, b):
            pltpu.async_copy(rows.at[b], acc.at[dstb.at[ch]], sems[b],
                             add=True)

        def wait_s(ch, b):
            pltpu.make_async_copy(rows.at[b], acc.at[dstb.at[ch]],
                                  sems[b]).wait()

        pltpu.async_copy(src_hbm.at[pl.ds(base + ANBUF, ANBUF)], srcs.at[1],
                         semi)
        for ch in range(ALAG):
            fire_g(ch, ch, srcs.at[0, ch])
        for ch in range(ALAG):
            fire_g(ch + ALAG, ch + ALAG, srcs.at[0, ch + ALAG])
            wait_g(ch)
            fire_s(ch, ch)

        def main(r, carry):
            gp = lax.rem(r + 1, 2)
            pltpu.make_async_copy(
                src_hbm.at[pl.ds(base, ANBUF)], srcs.at[0], semi).wait()
            for j in range(ANBUF):
                ch = ALAG + r * ANBUF + j
                bj = (j + ALAG) % ANBUF
                wait_s(ch - ALAG, j)
                fire_g(ch + ALAG, j, srcs.at[gp, j])
                wait_g(bj)
                fire_s(ch, bj)
                if j == 1:
                    nxt = jnp.minimum(r + 2, ANGRP - 1)
                    pltpu.async_copy(
                        src_hbm.at[pl.ds(base + nxt * ANBUF, ANBUF)],
                        srcs.at[lax.rem(r, 2)], semi)
            return carry

        n_main = (ACPT - 2 * ALAG) // ANBUF
        lax.fori_loop(0, n_main, main, 0)
        pltpu.make_async_copy(src_hbm.at[pl.ds(base, ANBUF)], srcs.at[0],
                              semi).wait()
        for ch in range(ALAG + n_main * ANBUF, ACPT):
            wait_s(ch - ALAG, (ch - ALAG) % ANBUF)
            wait_g(ch % ANBUF)
            fire_s(ch, ch % ANBUF)
        for ch in range(ACPT - ALAG, ACPT):
            wait_s(ch, ch % ANBUF)
        plsc.subcore_barrier()
        pltpu.sync_copy(acc.at[pl.ds(s * RPT, RPT)],
                        out_hbm.at[c, pl.ds(s * RPT, RPT)])

    return k(hs, src2d, dst2d)


def _tc1(degp, x_p, W1):
    def body(degp_ref, x_ref, w_ref, hs_ref, dis_ref):
        d16 = degp_ref[0] + degp_ref[1]
        deg = jnp.sum(d16, axis=1, keepdims=True) * (1.0 / DEG_W) + 1.0
        dis = lax.rsqrt(deg)
        h = jnp.dot(x_ref[...], w_ref[...],
                    preferred_element_type=jnp.float32) * dis
        hs_ref[...] = h
        dis_ref[...] = dis

    return pl.pallas_call(
        body,
        grid=(_G,),
        in_specs=[
            pl.BlockSpec((NC, _R, DEG_W), lambda r: (0, r, 0)),
            pl.BlockSpec((_R, D), lambda r: (r, 0)),
            pl.BlockSpec((D, D), lambda r: (0, 0)),
        ],
        out_specs=[
            pl.BlockSpec((_R, D), lambda r: (r, 0)),
            pl.BlockSpec((_R, 1), lambda r: (r, 0)),
        ],
        out_shape=[
            jax.ShapeDtypeStruct((N_PAD, D), jnp.float32),
            jax.ShapeDtypeStruct((N_PAD, 1), jnp.float32),
        ],
    )(degp, x_p, W1)


def _tc_mid(p, hs, dis, b, W2, first):
    def body(p_ref, hs_ref, dis_ref, b_ref, w_ref, out_ref):
        r = pl.program_id(0)
        dis_v = dis_ref[...]
        u = dis_v * (p_ref[0] + p_ref[1] + hs_ref[...]) + b_ref[...]
        if first:
            row = lax.broadcasted_iota(jnp.int32, (_R, 1), 0) + r * _R
            t = jnp.where(row < N, jnp.maximum(u, 0.0), 0.0)
            out_ref[...] = jnp.dot(
                t, w_ref[...], preferred_element_type=jnp.float32) * dis_v
        else:
            out_ref[...] = u

    return pl.pallas_call(
        body,
        grid=(_G,),
        in_specs=[
            pl.BlockSpec((NC, _R, D), lambda r: (0, r, 0)),
            pl.BlockSpec((_R, D), lambda r: (r, 0)),
            pl.BlockSpec((_R, 1), lambda r: (r, 0)),
            pl.BlockSpec((1, D), lambda r: (0, 0)),
            pl.BlockSpec((D, D), lambda r: (0, 0)),
        ],
        out_specs=pl.BlockSpec((_R, D), lambda r: (r, 0)),
        out_shape=jax.ShapeDtypeStruct((N_PAD, D), jnp.float32),
    )(p, hs, dis, b, W2)


def kernel(x, edge_index, W1, b1, W2, b2):
    src = edge_index[0]
    dst = edge_index[1]
    pad = jnp.full((E_PAD - E,), N, jnp.int32)
    src2d = jnp.concatenate([src, pad]).reshape(NCHUNKS, CHUNK)
    dst2d = jnp.concatenate([dst, pad]).reshape(NCHUNKS, CHUNK)
    x_p = jnp.pad(x, ((0, N_PAD - N), (0, 0)))

    src64 = src2d.reshape(E_PAD // ACH, ACH)
    dst64 = dst2d.reshape(E_PAD // ACH, ACH)
    degp = _sc_deg(dst2d)
    hs1, dis = _tc1(degp, x_p, W1)
    p = _sc_agg(hs1, src64, dst64)
    hs2 = _tc_mid(p, hs1, dis, b1.reshape(1, D), W2, first=True)
    q = _sc_agg(hs2, src64, dst64)
    out = _tc_mid(q, hs2, dis, b2.reshape(1, D), W2, first=False)
    return out[:N]


# speedup vs baseline: 1.3494x; 1.3449x over previous
import functools

import jax
import jax.numpy as jnp
from jax import lax
from jax.experimental import pallas as pl
from jax.experimental.pallas import tpu as pltpu
from jax.experimental.pallas import tpu_sc as plsc

N = 10000
D = 128
HD = D // 2
E = 320000

NC = 2
NS = 16
NW = NC * NS

CHUNK = 128
E_PAD = 327680
NCHUNKS = E_PAD // CHUNK
CPT_DEG = NCHUNKS // NW
CPT = NCHUNKS // NS
N_PAD = 10240
RPT = N_PAD // NS
DEG_W = 16
NBUF = 6
LAG = 3

_R = 2048
_G = N_PAD // _R


def _sc_mesh():
    return plsc.VectorSubcoreMesh(core_axis_name="c", subcore_axis_name="s")


_SC_PARAMS = pltpu.CompilerParams(use_tc_tiling_on_sc=False)


def _sc_deg(dst2d):
    @functools.partial(
        pl.kernel,
        mesh=_sc_mesh(),
        out_type=jax.ShapeDtypeStruct((NC, N_PAD, DEG_W), jnp.float32),
        scratch_types=[
            pltpu.VMEM((CPT_DEG, CHUNK), jnp.int32),
            pltpu.VMEM((CHUNK, DEG_W), jnp.float32),
            pltpu.VMEM((CHUNK, DEG_W), jnp.float32),
            pltpu.VMEM_SHARED((N_PAD, DEG_W), jnp.float32),
            pltpu.SemaphoreType.DMA,
        ],
        compiler_params=_SC_PARAMS,
    )
    def k(dst_hbm, out_hbm, idxb, onesb, zerob, acc, sem):
        c = lax.axis_index("c")
        s = lax.axis_index("s")
        w = c * NS + s

        def fill(i, carry):
            onesb[i, :] = jnp.full((DEG_W,), 1.0, jnp.float32)
            zerob[i, :] = jnp.zeros((DEG_W,), jnp.float32)
            return carry

        lax.fori_loop(0, CHUNK, fill, 0)
        for kk in range(RPT // CHUNK):
            pltpu.sync_copy(zerob, acc.at[pl.ds(s * RPT + kk * CHUNK, CHUNK)])
        pltpu.sync_copy(dst_hbm.at[pl.ds(w * CPT_DEG, CPT_DEG)], idxb)
        plsc.subcore_barrier()

        def fire(g, carry):
            pltpu.async_copy(onesb, acc.at[idxb.at[g]], sem, add=True)
            return carry

        lax.fori_loop(0, CPT_DEG, fire, 0)

        def drain(g, carry):
            pltpu.make_async_copy(onesb, acc.at[idxb.at[g]], sem).wait()
            return carry

        lax.fori_loop(0, CPT_DEG, drain, 0)
        plsc.subcore_barrier()
        pltpu.sync_copy(acc.at[pl.ds(s * RPT, RPT)],
                        out_hbm.at[c, pl.ds(s * RPT, RPT)])

    return k(dst2d)


def _sc_agg(hs_flat, src3d, dst2d):
    @functools.partial(
        pl.kernel,
        mesh=_sc_mesh(),
        out_type=jax.ShapeDtypeStruct((NC, N_PAD, HD), jnp.float32),
        scratch_types=[
            pltpu.VMEM((CPT, CHUNK), jnp.int32),
            pltpu.VMEM((CPT, CHUNK), jnp.int32),
            pltpu.VMEM((NBUF, CHUNK, HD), jnp.float32),
            pltpu.VMEM_SHARED((N_PAD, HD), jnp.float32),
            [pltpu.SemaphoreType.DMA] * NBUF,
        ],
        compiler_params=_SC_PARAMS,
    )
    def k(hs_hbm, src_hbm, dst_hbm, out_hbm, srcb, dstb, rows, acc, sems):
        c = lax.axis_index("c")
        s = lax.axis_index("s")

        def zrow(i, carry):
            for j in range(HD // 16):
                rows[0, i, pl.ds(j * 16, 16)] = jnp.zeros((16,), jnp.float32)
            return carry

        lax.fori_loop(0, CHUNK, zrow, 0)
        for kk in range(RPT // CHUNK):
            pltpu.sync_copy(rows.at[0],
                            acc.at[pl.ds(s * RPT + kk * CHUNK, CHUNK)])
        pltpu.sync_copy(src_hbm.at[c, pl.ds(s * CPT, CPT)], srcb)
        pltpu.sync_copy(dst_hbm.at[pl.ds(s * CPT, CPT)], dstb)
        plsc.subcore_barrier()

        def fire_g(ch, b):
            pltpu.async_copy(hs_hbm.at[srcb.at[ch]], rows.at[b], sems[b])

        def wait_g(ch, b):
            pltpu.make_async_copy(
                hs_hbm.at[srcb.at[ch]], rows.at[b], sems[b]).wait()

        def fire_s(ch, b):
            pltpu.async_copy(rows.at[b], acc.at[dstb.at[ch]], sems[b],
                             add=True)

        def wait_s(ch, b):
            pltpu.make_async_copy(rows.at[b], acc.at[dstb.at[ch]],
                                  sems[b]).wait()

        for ch in range(LAG):
            fire_g(ch, ch % NBUF)
        for ch in range(LAG):
            fire_g(ch + LAG, (ch + LAG) % NBUF)
            wait_g(ch, ch % NBUF)
            fire_s(ch, ch % NBUF)

        def main(r, carry):
            for j in range(NBUF):
                ch = LAG + r * NBUF + j
                bj = (j + LAG) % NBUF
                wait_s(ch - LAG, j)
                fire_g(ch + LAG, j)
                wait_g(ch, bj)
                fire_s(ch, bj)
            return carry

        n_main = (CPT - 2 * LAG) // NBUF
        lax.fori_loop(0, n_main, main, 0)
        for ch in range(LAG + n_main * NBUF, CPT):
            wait_s(ch - LAG, (ch - LAG) % NBUF)
            if ch + LAG < CPT:
                fire_g(ch + LAG, (ch + LAG) % NBUF)
            wait_g(ch, ch % NBUF)
            fire_s(ch, ch % NBUF)
        for ch in range(CPT - LAG, CPT):
            wait_s(ch, ch % NBUF)
        plsc.subcore_barrier()
        pltpu.sync_copy(acc.at[pl.ds(s * RPT, RPT)],
                        out_hbm.at[c, pl.ds(s * RPT, RPT)])

    return k(hs_flat, src3d, dst2d)


def _tc1(degp, x_p, W1):
    def body(degp_ref, x_ref, w_ref, hs_ref, dis_ref):
        d16 = degp_ref[0] + degp_ref[1]
        deg = jnp.sum(d16, axis=1, keepdims=True) * (1.0 / DEG_W) + 1.0
        dis = lax.rsqrt(deg)
        h = jnp.dot(x_ref[...], w_ref[...],
                    preferred_element_type=jnp.float32) * dis
        hs_ref[0, :, :] = h[:, :HD]
        hs_ref[1, :, :] = h[:, HD:]
        dis_ref[...] = dis

    return pl.pallas_call(
        body,
        grid=(_G,),
        in_specs=[
            pl.BlockSpec((NC, _R, DEG_W), lambda r: (0, r, 0)),
            pl.BlockSpec((_R, D), lambda r: (r, 0)),
            pl.BlockSpec((D, D), lambda r: (0, 0)),
        ],
        out_specs=[
            pl.BlockSpec((NC, _R, HD), lambda r: (0, r, 0)),
            pl.BlockSpec((_R, 1), lambda r: (r, 0)),
        ],
        out_shape=[
            jax.ShapeDtypeStruct((NC, N_PAD, HD), jnp.float32),
            jax.ShapeDtypeStruct((N_PAD, 1), jnp.float32),
        ],
    )(degp, x_p, W1)


def _tc2(p, hs1, dis, b1, W2):
    def body(p_ref, hs1_ref, dis_ref, b1_ref, w_ref, hs2_ref):
        r = pl.program_id(0)
        dis_v = dis_ref[...]
        row = lax.broadcasted_iota(jnp.int32, (_R, 1), 0) + r * _R
        keep = row < N
        t_lo = dis_v * (p_ref[0] + hs1_ref[0]) + b1_ref[:, :HD]
        t_hi = dis_v * (p_ref[1] + hs1_ref[1]) + b1_ref[:, HD:]
        t_lo = jnp.where(keep, jnp.maximum(t_lo, 0.0), 0.0)
        t_hi = jnp.where(keep, jnp.maximum(t_hi, 0.0), 0.0)
        h2 = (jnp.dot(t_lo, w_ref[:HD, :], preferred_element_type=jnp.float32)
              + jnp.dot(t_hi, w_ref[HD:, :],
                        preferred_element_type=jnp.float32)) * dis_v
        hs2_ref[0, :, :] = h2[:, :HD]
        hs2_ref[1, :, :] = h2[:, HD:]

    return pl.pallas_call(
        body,
        grid=(_G,),
        in_specs=[
            pl.BlockSpec((NC, _R, HD), lambda r: (0, r, 0)),
            pl.BlockSpec((NC, _R, HD), lambda r: (0, r, 0)),
            pl.BlockSpec((_R, 1), lambda r: (r, 0)),
            pl.BlockSpec((1, D), lambda r: (0, 0)),
            pl.BlockSpec((D, D), lambda r: (0, 0)),
        ],
        out_specs=pl.BlockSpec((NC, _R, HD), lambda r: (0, r, 0)),
        out_shape=jax.ShapeDtypeStruct((NC, N_PAD, HD), jnp.float32),
    )(p, hs1, dis, b1, W2)


def _tc3(q, hs2, dis, b2):
    def body(q_ref, hs2_ref, dis_ref, b2_ref, out_ref):
        dis_v = dis_ref[...]
        out_ref[:, :HD] = dis_v * (q_ref[0] + hs2_ref[0]) + b2_ref[:, :HD]
        out_ref[:, HD:] = dis_v * (q_ref[1] + hs2_ref[1]) + b2_ref[:, HD:]

    return pl.pallas_call(
        body,
        grid=(_G,),
        in_specs=[
            pl.BlockSpec((NC, _R, HD), lambda r: (0, r, 0)),
            pl.BlockSpec((NC, _R, HD), lambda r: (0, r, 0)),
            pl.BlockSpec((_R, 1), lambda r: (r, 0)),
            pl.BlockSpec((1, D), lambda r: (0, 0)),
        ],
        out_specs=pl.BlockSpec((_R, D), lambda r: (r, 0)),
        out_shape=jax.ShapeDtypeStruct((N_PAD, D), jnp.float32),
    )(q, hs2, dis, b2)


def kernel(x, edge_index, W1, b1, W2, b2):
    src = edge_index[0]
    dst = edge_index[1]
    pad = jnp.full((E_PAD - E,), N, jnp.int32)
    src2d = jnp.concatenate([src, pad]).reshape(NCHUNKS, CHUNK)
    dst2d = jnp.concatenate([dst, pad]).reshape(NCHUNKS, CHUNK)
    src3d = jnp.stack([src2d, src2d + N_PAD])
    x_p = jnp.pad(x, ((0, N_PAD - N), (0, 0)))

    degp = _sc_deg(dst2d)
    hs1, dis = _tc1(degp, x_p, W1)
    p = _sc_agg(hs1.reshape(NC * N_PAD, HD), src3d, dst2d)
    hs2 = _tc2(p, hs1, dis, b1.reshape(1, D), W2)
    q = _sc_agg(hs2.reshape(NC * N_PAD, HD), src3d, dst2d)
    out = _tc3(q, hs2, dis, b2.reshape(1, D))
    return out[:N]


# speedup vs baseline: 1.3600x; 1.0078x over previous
import functools

import jax
import jax.numpy as jnp
from jax import lax
from jax.experimental import pallas as pl
from jax.experimental.pallas import tpu as pltpu
from jax.experimental.pallas import tpu_sc as plsc

N = 10000
D = 128
HD = D // 2
E = 320000

NC = 2
NS = 16
NW = NC * NS

CHUNK = 128
E_PAD = 327680
NCHUNKS = E_PAD // CHUNK
CPT_DEG = NCHUNKS // NW
CPT = NCHUNKS // NS
N_PAD = 10240
RPT = N_PAD // NS
DEG_W = 16
NBUF = 6
LAG = 3

_R = 2048
_G = N_PAD // _R


def _sc_mesh():
    return plsc.VectorSubcoreMesh(core_axis_name="c", subcore_axis_name="s")


_SC_PARAMS = pltpu.CompilerParams(use_tc_tiling_on_sc=False)


def _sc_deg(dst2d):
    @functools.partial(
        pl.kernel,
        mesh=_sc_mesh(),
        out_type=jax.ShapeDtypeStruct((NC, N_PAD, DEG_W), jnp.float32),
        scratch_types=[
            pltpu.VMEM((CPT_DEG, CHUNK), jnp.int32),
            pltpu.VMEM((CHUNK, DEG_W), jnp.float32),
            pltpu.VMEM((CHUNK, DEG_W), jnp.float32),
            pltpu.VMEM_SHARED((N_PAD, DEG_W), jnp.float32),
            pltpu.SemaphoreType.DMA,
        ],
        compiler_params=_SC_PARAMS,
    )
    def k(dst_hbm, out_hbm, idxb, onesb, zerob, acc, sem):
        c = lax.axis_index("c")
        s = lax.axis_index("s")
        w = c * NS + s

        def fill(i, carry):
            onesb[i, :] = jnp.full((DEG_W,), 1.0, jnp.float32)
            zerob[i, :] = jnp.zeros((DEG_W,), jnp.float32)
            return carry

        lax.fori_loop(0, CHUNK, fill, 0)
        for kk in range(RPT // CHUNK):
            pltpu.sync_copy(zerob, acc.at[pl.ds(s * RPT + kk * CHUNK, CHUNK)])
        pltpu.sync_copy(dst_hbm.at[pl.ds(w * CPT_DEG, CPT_DEG)], idxb)
        plsc.subcore_barrier()

        def fire(g, carry):
            pltpu.async_copy(onesb, acc.at[idxb.at[g]], sem, add=True)
            return carry

        lax.fori_loop(0, CPT_DEG, fire, 0)

        def drain(g, carry):
            pltpu.make_async_copy(onesb, acc.at[idxb.at[g]], sem).wait()
            return carry

        lax.fori_loop(0, CPT_DEG, drain, 0)
        plsc.subcore_barrier()
        pltpu.sync_copy(acc.at[pl.ds(s * RPT, RPT)],
                        out_hbm.at[c, pl.ds(s * RPT, RPT)])

    return k(dst2d)


def _sc_agg(hs_flat, src3d, dst2d):
    @functools.partial(
        pl.kernel,
        mesh=_sc_mesh(),
        out_type=jax.ShapeDtypeStruct((NC, N_PAD, HD), jnp.float32),
        scratch_types=[
            pltpu.VMEM((CPT, CHUNK), jnp.int32),
            pltpu.VMEM((CPT, CHUNK), jnp.int32),
            pltpu.VMEM((NBUF, CHUNK, HD), jnp.float32),
            pltpu.VMEM_SHARED((N_PAD, HD), jnp.float32),
            [pltpu.SemaphoreType.DMA] * NBUF,
        ],
        compiler_params=_SC_PARAMS,
    )
    def k(hs_hbm, src_hbm, dst_hbm, out_hbm, srcb, dstb, rows, acc, sems):
        c = lax.axis_index("c")
        s = lax.axis_index("s")

        def zrow(i, carry):
            for j in range(HD // 16):
                rows[0, i, pl.ds(j * 16, 16)] = jnp.zeros((16,), jnp.float32)
            return carry

        lax.fori_loop(0, CHUNK, zrow, 0)
        for kk in range(RPT // CHUNK):
            pltpu.sync_copy(rows.at[0],
                            acc.at[pl.ds(s * RPT + kk * CHUNK, CHUNK)])
        pltpu.sync_copy(src_hbm.at[c, pl.ds(s * CPT, CPT)], srcb)
        pltpu.sync_copy(dst_hbm.at[pl.ds(s * CPT, CPT)], dstb)
        plsc.subcore_barrier()

        def fire_g(ch, b):
            pltpu.async_copy(hs_hbm.at[srcb.at[ch]], rows.at[b], sems[b])

        def wait_g(ch, b):
            pltpu.make_async_copy(
                hs_hbm.at[srcb.at[ch]], rows.at[b], sems[b]).wait()

        def fire_s(ch, b):
            pltpu.async_copy(rows.at[b], acc.at[dstb.at[ch]], sems[b],
                             add=True)

        def wait_s(ch, b):
            pltpu.make_async_copy(rows.at[b], acc.at[dstb.at[ch]],
                                  sems[b]).wait()

        for ch in range(LAG):
            fire_g(ch, ch % NBUF)
        for ch in range(LAG):
            fire_g(ch + LAG, (ch + LAG) % NBUF)
            wait_g(ch, ch % NBUF)
            fire_s(ch, ch % NBUF)

        def main(r, carry):
            for j in range(NBUF):
                ch = LAG + r * NBUF + j
                bj = (j + LAG) % NBUF
                wait_s(ch - LAG, j)
                fire_g(ch + LAG, j)
                wait_g(ch, bj)
                fire_s(ch, bj)
            return carry

        n_main = (CPT - 2 * LAG) // NBUF
        lax.fori_loop(0, n_main, main, 0)
        for ch in range(LAG + n_main * NBUF, CPT):
            wait_s(ch - LAG, (ch - LAG) % NBUF)
            if ch + LAG < CPT:
                fire_g(ch + LAG, (ch + LAG) % NBUF)
            wait_g(ch, ch % NBUF)
            fire_s(ch, ch % NBUF)
        for ch in range(CPT - LAG, CPT):
            wait_s(ch, ch % NBUF)
        plsc.subcore_barrier()
        pltpu.sync_copy(acc.at[pl.ds(s * RPT, RPT)],
                        out_hbm.at[c, pl.ds(s * RPT, RPT)])

    return k(hs_flat, src3d, dst2d)


def _tc1(degp, x_p, W1):
    def body(degp_ref, x_ref, w_ref, hs_ref, dis_ref):
        r = pl.program_id(0)
        d16 = degp_ref[0] + degp_ref[1]
        deg = jnp.sum(d16, axis=1, keepdims=True) * (1.0 / DEG_W) + 1.0
        dis = lax.rsqrt(deg)
        h = jnp.dot(x_ref[...], w_ref[...],
                    preferred_element_type=jnp.float32) * dis
        row = lax.broadcasted_iota(jnp.int32, (_R, 1), 0) + r * _R
        h = jnp.where(row < N, h, 0.0)
        hs_ref[0, :, :] = h[:, :HD]
        hs_ref[1, :, :] = h[:, HD:]
        dis_ref[...] = dis

    return pl.pallas_call(
        body,
        grid=(_G,),
        in_specs=[
            pl.BlockSpec((NC, _R, DEG_W), lambda r: (0, r, 0)),
            pl.BlockSpec((_R, D), lambda r: (r, 0)),
            pl.BlockSpec((D, D), lambda r: (0, 0)),
        ],
        out_specs=[
            pl.BlockSpec((NC, _R, HD), lambda r: (0, r, 0)),
            pl.BlockSpec((_R, 1), lambda r: (r, 0)),
        ],
        out_shape=[
            jax.ShapeDtypeStruct((NC, N_PAD, HD), jnp.float32),
            jax.ShapeDtypeStruct((N_PAD, 1), jnp.float32),
        ],
    )(degp, x_p, W1)


def _tc2(p, hs1, dis, b1, W2):
    def body(p_ref, hs1_ref, dis_ref, b1_ref, w_ref, hs2_ref):
        r = pl.program_id(0)
        dis_v = dis_ref[...]
        row = lax.broadcasted_iota(jnp.int32, (_R, 1), 0) + r * _R
        keep = row < N
        t_lo = dis_v * (p_ref[0] + hs1_ref[0]) + b1_ref[:, :HD]
        t_hi = dis_v * (p_ref[1] + hs1_ref[1]) + b1_ref[:, HD:]
        t_lo = jnp.where(keep, jnp.maximum(t_lo, 0.0), 0.0)
        t_hi = jnp.where(keep, jnp.maximum(t_hi, 0.0), 0.0)
        h2 = (jnp.dot(t_lo, w_ref[:HD, :], preferred_element_type=jnp.float32)
              + jnp.dot(t_hi, w_ref[HD:, :],
                        preferred_element_type=jnp.float32)) * dis_v
        hs2_ref[0, :, :] = h2[:, :HD]
        hs2_ref[1, :, :] = h2[:, HD:]

    return pl.pallas_call(
        body,
        grid=(_G,),
        in_specs=[
            pl.BlockSpec((NC, _R, HD), lambda r: (0, r, 0)),
            pl.BlockSpec((NC, _R, HD), lambda r: (0, r, 0)),
            pl.BlockSpec((_R, 1), lambda r: (r, 0)),
            pl.BlockSpec((1, D), lambda r: (0, 0)),
            pl.BlockSpec((D, D), lambda r: (0, 0)),
        ],
        out_specs=pl.BlockSpec((NC, _R, HD), lambda r: (0, r, 0)),
        out_shape=jax.ShapeDtypeStruct((NC, N_PAD, HD), jnp.float32),
    )(p, hs1, dis, b1, W2)


def _tc3(q, hs2, dis, b2):
    def body(q_ref, hs2_ref, dis_ref, b2_ref, out_ref):
        dis_v = dis_ref[...]
        out_ref[:, :HD] = dis_v * (q_ref[0] + hs2_ref[0]) + b2_ref[:, :HD]
        out_ref[:, HD:] = dis_v * (q_ref[1] + hs2_ref[1]) + b2_ref[:, HD:]

    return pl.pallas_call(
        body,
        grid=(_G,),
        in_specs=[
            pl.BlockSpec((NC, _R, HD), lambda r: (0, r, 0)),
            pl.BlockSpec((NC, _R, HD), lambda r: (0, r, 0)),
            pl.BlockSpec((_R, 1), lambda r: (r, 0)),
            pl.BlockSpec((1, D), lambda r: (0, 0)),
        ],
        out_specs=pl.BlockSpec((_R, D), lambda r: (r, 0)),
        out_shape=jax.ShapeDtypeStruct((N, D), jnp.float32),
    )(q, hs2, dis, b2)


def kernel(x, edge_index, W1, b1, W2, b2):
    src = edge_index[0]
    dst = edge_index[1]
    pad = jnp.full((E_PAD - E,), N, jnp.int32)
    src2d = jnp.concatenate([src, pad]).reshape(NCHUNKS, CHUNK)
    dst2d = jnp.concatenate([dst, pad]).reshape(NCHUNKS, CHUNK)
    src3d = jnp.stack([src2d, src2d + N_PAD])

    degp = _sc_deg(dst2d)
    hs1, dis = _tc1(degp, x, W1)
    p = _sc_agg(hs1.reshape(NC * N_PAD, HD), src3d, dst2d)
    hs2 = _tc2(p, hs1, dis, b1.reshape(1, D), W2)
    q = _sc_agg(hs2.reshape(NC * N_PAD, HD), src3d, dst2d)
    return _tc3(q, hs2, dis, b2.reshape(1, D))
